# baseline (speedup 1.0000x reference)
import jax
import jax.numpy as jnp
from jax import lax
from jax.experimental import pallas as pl
from jax.experimental.pallas import tpu as pltpu
from jax.experimental.pallas import tpu_sc as plsc

N = 2097152
N_CLASSES = 32
NC = 2
NS = 16
NW = NC * NS
PER_W = N // NW
CHUNK = 2048
NCHUNKS = PER_W // CHUNK
LANES = 16


def _body(in_hbm, out_hbm, in_v, pos_v, out_v):
    wid = lax.axis_index("s") * NC + lax.axis_index("c")
    base = wid * PER_W
    lane32 = lax.iota(jnp.int32, LANES) * N_CLASSES
    ones = jnp.full((LANES,), 1.0, jnp.float32)
    zeros = jnp.zeros((LANES,), jnp.float32)

    def zero(i, c):
        out_v[pl.ds(i * LANES, LANES)] = zeros
        return c

    lax.fori_loop(0, CHUNK * N_CLASSES // LANES, zero, 0)

    def chunk_body(k, c):
        pltpu.sync_copy(in_hbm.at[pl.ds(base + k * CHUNK, CHUNK)], in_v)

        def fill(j, cc):
            x = in_v[pl.ds(j * LANES, LANES)]
            idx = (x * 32.0).astype(jnp.int32)
            pos = j * (LANES * N_CLASSES) + lane32 + idx
            pos_v[pl.ds(j * LANES, LANES)] = pos
            plsc.store_scatter(out_v, [pos], ones)
            return cc

        lax.fori_loop(0, CHUNK // LANES, fill, 0)

        pltpu.sync_copy(
            out_v, out_hbm.at[pl.ds((base + k * CHUNK) * N_CLASSES, CHUNK * N_CLASSES)]
        )

        def clear(j, cc):
            pos = pos_v[pl.ds(j * LANES, LANES)]
            plsc.store_scatter(out_v, [pos], zeros)
            return cc

        lax.fori_loop(0, CHUNK // LANES, clear, 0)
        return c

    lax.fori_loop(0, NCHUNKS, chunk_body, 0)


def kernel(inputs):
    mesh = plsc.VectorSubcoreMesh(core_axis_name="c", subcore_axis_name="s")
    f = pl.kernel(
        _body,
        mesh=mesh,
        out_type=jax.ShapeDtypeStruct((N * N_CLASSES,), jnp.float32),
        compiler_params=pltpu.CompilerParams(needs_layout_passes=False),
        scratch_types=[
            pltpu.VMEM((CHUNK,), jnp.float32),
            pltpu.VMEM((CHUNK,), jnp.int32),
            pltpu.VMEM((CHUNK * N_CLASSES,), jnp.float32),
        ],
    )
    flat = f(inputs)
    return flat.reshape(N, N_CLASSES)


# speedup vs baseline: 1.0716x; 1.0716x over previous
import jax
import jax.numpy as jnp
from jax import lax
from jax.experimental import pallas as pl
from jax.experimental.pallas import tpu as pltpu
from jax.experimental.pallas import tpu_sc as plsc

N = 2097152
N_CLASSES = 32
NC = 2
NS = 16
NW = NC * NS
PER_W = N // NW
CHUNK = 1024
NCHUNKS = PER_W // CHUNK
LANES = 16
UNROLL = 4


def _body(in_hbm, out_hbm, in_v0, in_v1, pos_v0, pos_v1, out_v0, out_v1,
          isem0, isem1, osem0, osem1):
    wid = lax.axis_index("s") * NC + lax.axis_index("c")
    base = wid * PER_W
    lane32 = lax.iota(jnp.int32, LANES) * N_CLASSES
    ones = jnp.full((LANES,), 1.0, jnp.float32)
    zeros = jnp.zeros((LANES,), jnp.float32)
    bufs = ((in_v0, pos_v0, out_v0, isem0, osem0),
            (in_v1, pos_v1, out_v1, isem1, osem1))

    def zero(i, c):
        for u in range(UNROLL):
            o = i * (LANES * UNROLL) + u * LANES
            out_v0[pl.ds(o, LANES)] = zeros
            out_v1[pl.ds(o, LANES)] = zeros
        return c

    lax.fori_loop(0, CHUNK * N_CLASSES // (LANES * UNROLL), zero, 0)

    def start_in(k, in_v, isem):
        kk = lax.rem(k, NCHUNKS) if not isinstance(k, int) else k % NCHUNKS
        pltpu.async_copy(in_hbm.at[pl.ds(base + kk * CHUNK, CHUNK)], in_v, isem)

    def wait_in(in_v, isem):
        pltpu.make_async_copy(in_hbm.at[pl.ds(base, CHUNK)], in_v, isem).wait()

    def wait_out(out_v, osem):
        pltpu.make_async_copy(
            out_v, out_hbm.at[pl.ds(base * N_CLASSES, CHUNK * N_CLASSES)], osem
        ).wait()

    def start_out(k, out_v, osem):
        pltpu.async_copy(
            out_v,
            out_hbm.at[pl.ds((base + k * CHUNK) * N_CLASSES, CHUNK * N_CLASSES)],
            osem,
        )

    def fill(in_v, pos_v, out_v):
        def body(j, c):
            for u in range(UNROLL):
                o = j * (LANES * UNROLL) + u * LANES
                x = in_v[pl.ds(o, LANES)]
                idx = (x * 32.0).astype(jnp.int32)
                pos = o * N_CLASSES + lane32 + idx
                pos_v[pl.ds(o, LANES)] = pos
                plsc.store_scatter(out_v, [pos], ones)
            return c

        lax.fori_loop(0, CHUNK // (LANES * UNROLL), body, 0)

    def clear(pos_v, out_v):
        def body(j, c):
            for u in range(UNROLL):
                o = j * (LANES * UNROLL) + u * LANES
                pos = pos_v[pl.ds(o, LANES)]
                plsc.store_scatter(out_v, [pos], zeros)
            return c

        lax.fori_loop(0, CHUNK // (LANES * UNROLL), body, 0)

    start_in(0, in_v0, isem0)
    start_in(1, in_v1, isem1)

    for b in range(2):
        in_v, pos_v, out_v, isem, osem = bufs[b]
        wait_in(in_v, isem)
        fill(in_v, pos_v, out_v)
        start_out(b, out_v, osem)
        start_in(b + 2, in_v, isem)

    def group(g, c):
        for b in range(2):
            in_v, pos_v, out_v, isem, osem = bufs[b]
            k = 2 * g + b
            wait_in(in_v, isem)
            wait_out(out_v, osem)
            clear(pos_v, out_v)
            fill(in_v, pos_v, out_v)
            start_out(k, out_v, osem)
            start_in(k + 2, in_v, isem)
        return c

    lax.fori_loop(1, NCHUNKS // 2, group, 0)

    for b in range(2):
        in_v, pos_v, out_v, isem, osem = bufs[b]
        wait_in(in_v, isem)
        wait_out(out_v, osem)


def kernel(inputs):
    mesh = plsc.VectorSubcoreMesh(core_axis_name="c", subcore_axis_name="s")
    f = pl.kernel(
        _body,
        mesh=mesh,
        out_type=jax.ShapeDtypeStruct((N * N_CLASSES,), jnp.float32),
        compiler_params=pltpu.CompilerParams(needs_layout_passes=False),
        scratch_types=[
            pltpu.VMEM((CHUNK,), jnp.float32),
            pltpu.VMEM((CHUNK,), jnp.float32),
            pltpu.VMEM((CHUNK,), jnp.int32),
            pltpu.VMEM((CHUNK,), jnp.int32),
            pltpu.VMEM((CHUNK * N_CLASSES,), jnp.float32),
            pltpu.VMEM((CHUNK * N_CLASSES,), jnp.float32),
            pltpu.SemaphoreType.DMA,
            pltpu.SemaphoreType.DMA,
            pltpu.SemaphoreType.DMA,
            pltpu.SemaphoreType.DMA,
        ],
    )
    flat = f(inputs)
    return flat.reshape(N, N_CLASSES)


# speedup vs baseline: 1.0720x; 1.0004x over previous
import jax
import jax.numpy as jnp
from jax import lax
from jax.experimental import pallas as pl
from jax.experimental.pallas import tpu as pltpu
from jax.experimental.pallas import tpu_sc as plsc

N = 2097152
N_CLASSES = 32
NC = 2
NS = 16
NW = NC * NS
PER_W = N // NW
CHUNK = 1024
NCHUNKS = PER_W // CHUNK
LANES = 16
UNROLL = 4


def _body(in_hbm, out_hbm, in_v0, in_v1, bin_v0, bin_v1, out_v0, out_v1,
          isem0, isem1, osem0, osem1):
    wid = lax.axis_index("s") * NC + lax.axis_index("c")
    base = wid * PER_W
    lane = lax.iota(jnp.int32, LANES)
    ones = jnp.full((LANES,), 1.0, jnp.float32)
    zeros = jnp.zeros((LANES,), jnp.float32)
    bufs = ((in_v0, bin_v0, out_v0, isem0, osem0),
            (in_v1, bin_v1, out_v1, isem1, osem1))

    def zero(i, c):
        for u in range(UNROLL):
            r = i * UNROLL + u
            out_v0[r, pl.ds(0, LANES)] = zeros
            out_v0[r, pl.ds(LANES, LANES)] = zeros
            out_v1[r, pl.ds(0, LANES)] = zeros
            out_v1[r, pl.ds(LANES, LANES)] = zeros
        return c

    lax.fori_loop(0, CHUNK // UNROLL, zero, 0)

    def start_in(k, in_v, isem):
        kk = lax.rem(k, NCHUNKS) if not isinstance(k, int) else k % NCHUNKS
        pltpu.async_copy(in_hbm.at[pl.ds(base + kk * CHUNK, CHUNK)], in_v, isem)

    def wait_in(in_v, isem):
        pltpu.make_async_copy(in_hbm.at[pl.ds(base, CHUNK)], in_v, isem).wait()

    def wait_out(out_v, osem):
        pltpu.make_async_copy(out_v, out_hbm.at[pl.ds(base, CHUNK)], osem).wait()

    def start_out(k, out_v, osem):
        pltpu.async_copy(out_v, out_hbm.at[pl.ds(base + k * CHUNK, CHUNK)], osem)

    def fill(in_v, bin_v, out_v):
        def body(j, c):
            for u in range(UNROLL):
                o = j * (LANES * UNROLL) + u * LANES
                x = in_v[pl.ds(o, LANES)]
                idx = (x * 32.0).astype(jnp.int32)
                row = o + lane
                bin_v[pl.ds(o, LANES)] = idx
                plsc.store_scatter(out_v, [row, idx], ones)
            return c

        lax.fori_loop(0, CHUNK // (LANES * UNROLL), body, 0)

    def clear(bin_v, out_v):
        def body(j, c):
            for u in range(UNROLL):
                o = j * (LANES * UNROLL) + u * LANES
                idx = bin_v[pl.ds(o, LANES)]
                row = o + lane
                plsc.store_scatter(out_v, [row, idx], zeros)
            return c

        lax.fori_loop(0, CHUNK // (LANES * UNROLL), body, 0)

    start_in(0, in_v0, isem0)
    start_in(1, in_v1, isem1)

    for b in range(2):
        in_v, bin_v, out_v, isem, osem = bufs[b]
        wait_in(in_v, isem)
        fill(in_v, bin_v, out_v)
        start_out(b, out_v, osem)
        start_in(b + 2, in_v, isem)

    def group(g, c):
        for b in range(2):
            in_v, bin_v, out_v, isem, osem = bufs[b]
            k = 2 * g + b
            wait_in(in_v, isem)
            wait_out(out_v, osem)
            clear(bin_v, out_v)
            fill(in_v, bin_v, out_v)
            start_out(k, out_v, osem)
            start_in(k + 2, in_v, isem)
        return c

    lax.fori_loop(1, NCHUNKS // 2, group, 0)

    for b in range(2):
        in_v, bin_v, out_v, isem, osem = bufs[b]
        wait_in(in_v, isem)
        wait_out(out_v, osem)


def kernel(inputs):
    mesh = plsc.VectorSubcoreMesh(core_axis_name="c", subcore_axis_name="s")
    f = pl.kernel(
        _body,
        mesh=mesh,
        out_type=jax.ShapeDtypeStruct((N, N_CLASSES), jnp.float32),
        compiler_params=pltpu.CompilerParams(
            needs_layout_passes=False, use_tc_tiling_on_sc=False
        ),
        scratch_types=[
            pltpu.VMEM((CHUNK,), jnp.float32),
            pltpu.VMEM((CHUNK,), jnp.float32),
            pltpu.VMEM((CHUNK,), jnp.int32),
            pltpu.VMEM((CHUNK,), jnp.int32),
            pltpu.VMEM((CHUNK, N_CLASSES), jnp.float32),
            pltpu.VMEM((CHUNK, N_CLASSES), jnp.float32),
            pltpu.SemaphoreType.DMA,
            pltpu.SemaphoreType.DMA,
            pltpu.SemaphoreType.DMA,
            pltpu.SemaphoreType.DMA,
        ],
    )
    return f(inputs)


# speedup vs baseline: 10.3536x; 9.6581x over previous
import jax
import jax.numpy as jnp
from jax import lax
from jax.experimental import pallas as pl
from jax.experimental.pallas import tpu as pltpu
from jax.experimental.pallas import tpu_sc as plsc

N = 2097152
N_CLASSES = 32
NC = 2
NS = 16
NW = NC * NS
PER_W = N // NW
CN = 1024
NCHUNKS = PER_W // CN
LANES = 16
UNROLL = 4


def _body(in_hbm, out_hbm, in_v0, in_v1, bin_v0, bin_v1, out_v0, out_v1,
          isem0, isem1, osem0, osem1):
    wid = lax.axis_index("s") * NC + lax.axis_index("c")
    base = wid * PER_W
    lane = lax.iota(jnp.int32, LANES)
    ones = jnp.full((LANES,), 1.0, jnp.float32)
    zeros = jnp.zeros((LANES,), jnp.float32)
    bufs = ((in_v0, bin_v0, out_v0, isem0, osem0),
            (in_v1, bin_v1, out_v1, isem1, osem1))

    def zero(i, c):
        for r in range(N_CLASSES):
            out_v0[r, pl.ds(i * LANES, LANES)] = zeros
            out_v1[r, pl.ds(i * LANES, LANES)] = zeros
        return c

    lax.fori_loop(0, CN // LANES, zero, 0)

    def start_in(k, in_v, isem):
        kk = lax.rem(k, NCHUNKS) if not isinstance(k, int) else k % NCHUNKS
        pltpu.async_copy(in_hbm.at[pl.ds(base + kk * CN, CN)], in_v, isem)

    def wait_in(in_v, isem):
        pltpu.make_async_copy(in_hbm.at[pl.ds(base, CN)], in_v, isem).wait()

    def wait_out(out_v, osem):
        pltpu.make_async_copy(out_v, out_hbm.at[:, pl.ds(base, CN)], osem).wait()

    def start_out(k, out_v, osem):
        pltpu.async_copy(out_v, out_hbm.at[:, pl.ds(base + k * CN, CN)], osem)

    def fill(in_v, bin_v, out_v):
        def body(j, c):
            for u in range(UNROLL):
                o = j * (LANES * UNROLL) + u * LANES
                x = in_v[pl.ds(o, LANES)]
                idx = (x * 32.0).astype(jnp.int32)
                col = o + lane
                bin_v[pl.ds(o, LANES)] = idx
                plsc.store_scatter(out_v, [idx, col], ones)
            return c

        lax.fori_loop(0, CN // (LANES * UNROLL), body, 0)

    def clear(bin_v, out_v):
        def body(j, c):
            for u in range(UNROLL):
                o = j * (LANES * UNROLL) + u * LANES
                idx = bin_v[pl.ds(o, LANES)]
                col = o + lane
                plsc.store_scatter(out_v, [idx, col], zeros)
            return c

        lax.fori_loop(0, CN // (LANES * UNROLL), body, 0)

    start_in(0, in_v0, isem0)
    start_in(1, in_v1, isem1)

    for b in range(2):
        in_v, bin_v, out_v, isem, osem = bufs[b]
        wait_in(in_v, isem)
        fill(in_v, bin_v, out_v)
        start_out(b, out_v, osem)
        start_in(b + 2, in_v, isem)

    def group(g, c):
        for b in range(2):
            in_v, bin_v, out_v, isem, osem = bufs[b]
            k = 2 * g + b
            wait_in(in_v, isem)
            wait_out(out_v, osem)
            clear(bin_v, out_v)
            fill(in_v, bin_v, out_v)
            start_out(k, out_v, osem)
            start_in(k + 2, in_v, isem)
        return c

    lax.fori_loop(1, NCHUNKS // 2, group, 0)

    for b in range(2):
        in_v, bin_v, out_v, isem, osem = bufs[b]
        wait_in(in_v, isem)
        wait_out(out_v, osem)


def kernel(inputs):
    mesh = plsc.VectorSubcoreMesh(core_axis_name="c", subcore_axis_name="s")
    f = pl.kernel(
        _body,
        mesh=mesh,
        out_type=jax.ShapeDtypeStruct((N_CLASSES, N), jnp.float32),
        compiler_params=pltpu.CompilerParams(
            needs_layout_passes=False, use_tc_tiling_on_sc=True
        ),
        scratch_types=[
            pltpu.VMEM((CN,), jnp.float32),
            pltpu.VMEM((CN,), jnp.float32),
            pltpu.VMEM((CN,), jnp.int32),
            pltpu.VMEM((CN,), jnp.int32),
            pltpu.VMEM((N_CLASSES, CN), jnp.float32),
            pltpu.VMEM((N_CLASSES, CN), jnp.float32),
            pltpu.SemaphoreType.DMA,
            pltpu.SemaphoreType.DMA,
            pltpu.SemaphoreType.DMA,
            pltpu.SemaphoreType.DMA,
        ],
    )
    return jnp.transpose(f(inputs))
